# Initial kernel scaffold; baseline (speedup 1.0000x reference)
import jax
import jax.numpy as jnp
from jax.experimental import pallas as pl


def kernel(x, edge_index, edge_attr, W_e1, b_e1, W_e2, b_e2, Wq, bq, Wk, bk, Wv, bv, We, Wskip, bskip, Wbeta):
    raise NotImplementedError("write your pallas kernel here")


# speedup vs baseline: 8.9290x; 8.9290x over previous
import functools

import jax
import jax.numpy as jnp
from jax import lax
from jax.experimental import pallas as pl
from jax.experimental.pallas import tpu as pltpu
from jax.experimental.pallas import tpu_sc as plsc

_ROW = 144
_CHUNK = 16
_NC = 2
_NS = 16
_INV_SQRT_C = 1.0 / 5.656854249492381


def _fold_body(we2_ref, we_ref, be2_ref, w2e_ref, b2e_ref):
    w2e_ref[...] = jnp.dot(we2_ref[...], we_ref[...],
                           preferred_element_type=jnp.float32)
    b2e_ref[...] = jnp.dot(be2_ref[...], we_ref[...],
                           preferred_element_type=jnp.float32)


def _node_body(x_ref, wq_ref, bq_ref, wk_ref, bk_ref, wv_ref, bv_ref,
               ws_ref, bs_ref, q_ref, k_ref, v_ref, xr_ref):
    xb = x_ref[...]
    q_ref[...] = jnp.dot(xb, wq_ref[...], preferred_element_type=jnp.float32) + bq_ref[...]
    k_ref[...] = jnp.dot(xb, wk_ref[...], preferred_element_type=jnp.float32) + bk_ref[...]
    v_ref[...] = jnp.dot(xb, wv_ref[...], preferred_element_type=jnp.float32) + bv_ref[...]
    xr_ref[...] = jnp.dot(xb, ws_ref[...], preferred_element_type=jnp.float32) + bs_ref[...]


def _edge_body(ea_ref, w1_ref, b1_ref, w2e_ref, b2e_ref, e_ref):
    h = jnp.dot(ea_ref[...], w1_ref[...],
                preferred_element_type=jnp.float32) + b1_ref[...]
    h = jnp.where(h >= 0, h, 0.15 * h)
    e_ref[...] = jnp.dot(h, w2e_ref[...],
                         preferred_element_type=jnp.float32) + b2e_ref[...]


def _final_body(m0_ref, m1_ref, d0_ref, d1_ref, xr_ref, wb_ref, o_ref):
    m = m0_ref[...] + m1_ref[...]
    d4 = d0_ref[:, 0:4] + d1_ref[:, 0:4]
    r = lax.broadcasted_iota(jnp.int32, (4, 128), 0)
    cc = lax.broadcasted_iota(jnp.int32, (4, 128), 1) // 32
    sel = (r == cc).astype(jnp.float32)
    den = jnp.dot(d4, sel, preferred_element_type=jnp.float32)
    out = m / (den + 1e-16)
    xrb = xr_ref[...]
    wa = wb_ref[0:128, :] + wb_ref[256:384, :]
    wb = wb_ref[128:256, :] - wb_ref[256:384, :]
    z = (jnp.dot(out, wa, preferred_element_type=jnp.float32)
         + jnp.dot(xrb, wb, preferred_element_type=jnp.float32))
    beta = jax.nn.sigmoid(z)
    o_ref[...] = beta * xrb + (1.0 - beta) * out


def _sc_body(n_nodes, n_edges, q_hbm, k_hbm, v_hbm, e_hbm, src_hbm, dst_hbm,
             m0_hbm, m1_hbm, d0_hbm, d1_hbm, idx_s, idx_d, qr, kr, vr, er,
             msg, acc):
    c = lax.axis_index("c")
    s = lax.axis_index("s")
    wid = c * _NS + s

    rows_per_sub = n_nodes // _NS
    zv = jnp.zeros((16,), jnp.float32)

    def _zb(b, _):
        for j in range(_ROW // 16):
            msg[b, pl.ds(16 * j, 16)] = zv
        return 0

    lax.fori_loop(0, _CHUNK, _zb, 0)
    zrows = 5

    def _zcopy(t, _):
        pltpu.sync_copy(msg.at[pl.ds(0, zrows), :],
                        acc.at[pl.ds(s * rows_per_sub + t * zrows, zrows), :])
        return 0

    lax.fori_loop(0, rows_per_sub // zrows, _zcopy, 0)
    plsc.subcore_barrier()

    per_tile = n_edges // (_NC * _NS)
    base0 = wid * per_tile
    lane = jnp.arange(16, dtype=jnp.int32)

    def _chunk(i, _):
        base = base0 + i * _CHUNK
        pltpu.sync_copy(src_hbm.at[pl.ds(base, _CHUNK)], idx_s)
        pltpu.sync_copy(dst_hbm.at[pl.ds(base, _CHUNK)], idx_d)
        pltpu.sync_copy(e_hbm.at[pl.ds(base, _CHUNK), :], er)
        pltpu.sync_copy(k_hbm.at[idx_s], kr)
        pltpu.sync_copy(v_hbm.at[idx_s], vr)
        pltpu.sync_copy(q_hbm.at[idx_d], qr)

        def _edge(b, _):
            ev = [er[b, pl.ds(16 * j, 16)] for j in range(8)]
            pv = [qr[b, pl.ds(16 * j, 16)] * (kr[b, pl.ds(16 * j, 16)] + ev[j])
                  for j in range(8)]
            exl = zv
            for h in range(4):
                tot = jnp.sum(pv[2 * h] + pv[2 * h + 1]) * _INV_SQRT_C
                exv = jnp.exp(jnp.full((16,), tot, jnp.float32))
                for j2 in range(2):
                    jj = 2 * h + j2
                    msg[b, pl.ds(16 * jj, 16)] = (
                        vr[b, pl.ds(16 * jj, 16)] + ev[jj]) * exv
                exl = jnp.where(lane == h, exv, exl)
            msg[b, pl.ds(128, 16)] = exl
            return 0

        lax.fori_loop(0, _CHUNK, _edge, 0)
        pltpu.sync_copy(msg, acc.at[idx_d], add=True)
        return 0

    lax.fori_loop(0, per_tile // _CHUNK, _chunk, 0)
    plsc.subcore_barrier()

    r0 = s * rows_per_sub

    @pl.when(c == 0)
    def _():
        pltpu.sync_copy(acc.at[pl.ds(r0, rows_per_sub), pl.ds(0, 128)],
                        m0_hbm.at[pl.ds(r0, rows_per_sub), :])
        pltpu.sync_copy(acc.at[pl.ds(r0, rows_per_sub), pl.ds(128, 16)],
                        d0_hbm.at[pl.ds(r0, rows_per_sub), pl.ds(0, 16)])

    @pl.when(c == 1)
    def _():
        pltpu.sync_copy(acc.at[pl.ds(r0, rows_per_sub), pl.ds(0, 128)],
                        m1_hbm.at[pl.ds(r0, rows_per_sub), :])
        pltpu.sync_copy(acc.at[pl.ds(r0, rows_per_sub), pl.ds(128, 16)],
                        d1_hbm.at[pl.ds(r0, rows_per_sub), pl.ds(0, 16)])


def kernel(x, edge_index, edge_attr, W_e1, b_e1, W_e2, b_e2, Wq, bq, Wk, bk,
           Wv, bv, We, Wskip, bskip, Wbeta):
    n, d = x.shape
    e_cnt = edge_attr.shape[0]
    hc = Wq.shape[1]


    fold = pl.pallas_call(
        _fold_body,
        out_shape=(jax.ShapeDtypeStruct((d, d), jnp.float32),
                   jax.ShapeDtypeStruct((1, d), jnp.float32)),
    )
    w2e, b2e = fold(W_e2, We, b_e2.reshape(1, d))

    nblk = 2000
    node = pl.pallas_call(
        _node_body,
        grid=(n // nblk,),
        in_specs=[pl.BlockSpec((nblk, d), lambda i: (i, 0))] +
                 [pl.BlockSpec((d, hc), lambda i: (0, 0)),
                  pl.BlockSpec((1, hc), lambda i: (0, 0))] * 4,
        out_specs=[pl.BlockSpec((nblk, hc), lambda i: (i, 0))] * 4,
        out_shape=[jax.ShapeDtypeStruct((n, hc), jnp.float32)] * 4,
    )
    q, k, v, xr = node(x, Wq, bq.reshape(1, hc), Wk, bk.reshape(1, hc),
                       Wv, bv.reshape(1, hc), Wskip, bskip.reshape(1, hc))

    eblk = 4000
    edim = edge_attr.shape[1]
    edge = pl.pallas_call(
        _edge_body,
        grid=(e_cnt // eblk,),
        in_specs=[pl.BlockSpec((eblk, edim), lambda i: (i, 0)),
                  pl.BlockSpec((edim, d), lambda i: (0, 0)),
                  pl.BlockSpec((1, d), lambda i: (0, 0)),
                  pl.BlockSpec((d, d), lambda i: (0, 0)),
                  pl.BlockSpec((1, d), lambda i: (0, 0))],
        out_specs=pl.BlockSpec((eblk, d), lambda i: (i, 0)),
        out_shape=jax.ShapeDtypeStruct((e_cnt, d), jnp.float32),
    )
    e = edge(edge_attr, W_e1, b_e1.reshape(1, d), w2e, b2e)

    src = edge_index[0].astype(jnp.int32)
    dst = edge_index[1].astype(jnp.int32)

    mesh = plsc.VectorSubcoreMesh(core_axis_name="c", subcore_axis_name="s")
    sc = pl.kernel(
        functools.partial(_sc_body, n, e_cnt),
        out_type=(jax.ShapeDtypeStruct((n, d), jnp.float32),
                  jax.ShapeDtypeStruct((n, d), jnp.float32),
                  jax.ShapeDtypeStruct((n, d), jnp.float32),
                  jax.ShapeDtypeStruct((n, d), jnp.float32)),
        mesh=mesh,
        scratch_types=[
            pltpu.VMEM((_CHUNK,), jnp.int32),
            pltpu.VMEM((_CHUNK,), jnp.int32),
            pltpu.VMEM((_CHUNK, d), jnp.float32),
            pltpu.VMEM((_CHUNK, d), jnp.float32),
            pltpu.VMEM((_CHUNK, d), jnp.float32),
            pltpu.VMEM((_CHUNK, d), jnp.float32),
            pltpu.VMEM((_CHUNK, _ROW), jnp.float32),
            pltpu.VMEM_SHARED((n, _ROW), jnp.float32),
        ],
        compiler_params=pltpu.CompilerParams(use_tc_tiling_on_sc=False,
                                             needs_layout_passes=False),
    )
    m0, m1, d0, d1 = sc(q, k, v, e, src, dst)

    final = pl.pallas_call(
        _final_body,
        grid=(n // nblk,),
        in_specs=[pl.BlockSpec((nblk, hc), lambda i: (i, 0)),
                  pl.BlockSpec((nblk, hc), lambda i: (i, 0)),
                  pl.BlockSpec((nblk, hc), lambda i: (i, 0)),
                  pl.BlockSpec((nblk, hc), lambda i: (i, 0)),
                  pl.BlockSpec((nblk, hc), lambda i: (i, 0)),
                  pl.BlockSpec((3 * hc, 1), lambda i: (0, 0))],
        out_specs=pl.BlockSpec((nblk, hc), lambda i: (i, 0)),
        out_shape=jax.ShapeDtypeStruct((n, hc), jnp.float32),
    )
    return final(m0, m1, d0, d1, xr, Wbeta)


# speedup vs baseline: 13.5553x; 1.5181x over previous
import functools

import jax
import jax.numpy as jnp
from jax import lax
from jax.experimental import pallas as pl
from jax.experimental.pallas import tpu as pltpu
from jax.experimental.pallas import tpu_sc as plsc

_ROW = 144
_CHUNK = 16
_NC = 2
_NS = 16
_INV_SQRT_C = 1.0 / 5.656854249492381


def _fold_body(we2_ref, we_ref, be2_ref, w2e_ref, b2e_ref):
    w2e_ref[...] = jnp.dot(we2_ref[...], we_ref[...],
                           preferred_element_type=jnp.float32)
    b2e_ref[...] = jnp.dot(be2_ref[...], we_ref[...],
                           preferred_element_type=jnp.float32)


def _node_body(x_ref, wq_ref, bq_ref, wk_ref, bk_ref, wv_ref, bv_ref,
               ws_ref, bs_ref, q_ref, k_ref, v_ref, xr_ref):
    xb = x_ref[...]
    q_ref[...] = jnp.dot(xb, wq_ref[...], preferred_element_type=jnp.float32) + bq_ref[...]
    k_ref[...] = jnp.dot(xb, wk_ref[...], preferred_element_type=jnp.float32) + bk_ref[...]
    v_ref[...] = jnp.dot(xb, wv_ref[...], preferred_element_type=jnp.float32) + bv_ref[...]
    xr_ref[...] = jnp.dot(xb, ws_ref[...], preferred_element_type=jnp.float32) + bs_ref[...]


def _edge_body(ea_ref, w1_ref, b1_ref, w2e_ref, b2e_ref, e_ref):
    h = jnp.dot(ea_ref[...], w1_ref[...],
                preferred_element_type=jnp.float32) + b1_ref[...]
    h = jnp.where(h >= 0, h, 0.15 * h)
    e_ref[...] = jnp.dot(h, w2e_ref[...],
                         preferred_element_type=jnp.float32) + b2e_ref[...]


def _final_body(m0_ref, m1_ref, d0_ref, d1_ref, xr_ref, wb_ref, o_ref):
    m = m0_ref[...] + m1_ref[...]
    d4 = d0_ref[:, 0:4] + d1_ref[:, 0:4]
    r = lax.broadcasted_iota(jnp.int32, (4, 128), 0)
    cc = lax.broadcasted_iota(jnp.int32, (4, 128), 1) // 32
    sel = (r == cc).astype(jnp.float32)
    den = jnp.dot(d4, sel, preferred_element_type=jnp.float32)
    out = m / (den + 1e-16)
    xrb = xr_ref[...]
    wa = wb_ref[0:128, :] + wb_ref[256:384, :]
    wb = wb_ref[128:256, :] - wb_ref[256:384, :]
    z = (jnp.dot(out, wa, preferred_element_type=jnp.float32)
         + jnp.dot(xrb, wb, preferred_element_type=jnp.float32))
    beta = jax.nn.sigmoid(z)
    o_ref[...] = beta * xrb + (1.0 - beta) * out


def _sc_body(n_nodes, n_edges, q_hbm, k_hbm, v_hbm, e_hbm, src_hbm, dst_hbm,
             m0_hbm, m1_hbm, d0_hbm, d1_hbm, idx_s, idx_d, qr, kr, vr, er,
             msg, acc, sem_e, sem_k, sem_v, sem_q):
    c = lax.axis_index("c")
    s = lax.axis_index("s")
    wid = c * _NS + s

    rows_per_sub = n_nodes // _NS
    zv = jnp.zeros((16,), jnp.float32)

    def _zb(b, _):
        for j in range(_ROW // 16):
            msg[b, pl.ds(16 * j, 16)] = zv
        return 0

    lax.fori_loop(0, _CHUNK, _zb, 0)
    zrows = 5

    def _zcopy(t, _):
        pltpu.sync_copy(msg.at[pl.ds(0, zrows), :],
                        acc.at[pl.ds(s * rows_per_sub + t * zrows, zrows), :])
        return 0

    lax.fori_loop(0, rows_per_sub // zrows, _zcopy, 0)
    plsc.subcore_barrier()

    per_tile = n_edges // (_NC * _NS)
    base0 = wid * per_tile
    lane = jnp.arange(16, dtype=jnp.int32)

    def _chunk(i, _):
        base = base0 + i * _CHUNK
        cp_e = pltpu.async_copy(e_hbm.at[pl.ds(base, _CHUNK), :], er, sem_e)
        pltpu.sync_copy(src_hbm.at[pl.ds(base, _CHUNK)], idx_s)
        pltpu.sync_copy(dst_hbm.at[pl.ds(base, _CHUNK)], idx_d)
        cp_k = pltpu.async_copy(k_hbm.at[idx_s], kr, sem_k)
        cp_v = pltpu.async_copy(v_hbm.at[idx_s], vr, sem_v)
        cp_q = pltpu.async_copy(q_hbm.at[idx_d], qr, sem_q)
        cp_e.wait()
        cp_k.wait()
        cp_v.wait()
        cp_q.wait()

        def _edge(b, _):
            ev = [er[b, pl.ds(16 * j, 16)] for j in range(8)]
            pv = [qr[b, pl.ds(16 * j, 16)] * (kr[b, pl.ds(16 * j, 16)] + ev[j])
                  for j in range(8)]
            exl = zv
            for h in range(4):
                tot = jnp.sum(pv[2 * h] + pv[2 * h + 1]) * _INV_SQRT_C
                exv = jnp.exp(jnp.full((16,), tot, jnp.float32))
                for j2 in range(2):
                    jj = 2 * h + j2
                    msg[b, pl.ds(16 * jj, 16)] = (
                        vr[b, pl.ds(16 * jj, 16)] + ev[jj]) * exv
                exl = jnp.where(lane == h, exv, exl)
            msg[b, pl.ds(128, 16)] = exl
            return 0

        lax.fori_loop(0, _CHUNK, _edge, 0)
        pltpu.sync_copy(msg, acc.at[idx_d], add=True)
        return 0

    lax.fori_loop(0, per_tile // _CHUNK, _chunk, 0)
    plsc.subcore_barrier()

    r0 = s * rows_per_sub

    @pl.when(c == 0)
    def _():
        pltpu.sync_copy(acc.at[pl.ds(r0, rows_per_sub), pl.ds(0, 128)],
                        m0_hbm.at[pl.ds(r0, rows_per_sub), :])
        pltpu.sync_copy(acc.at[pl.ds(r0, rows_per_sub), pl.ds(128, 16)],
                        d0_hbm.at[pl.ds(r0, rows_per_sub), pl.ds(0, 16)])

    @pl.when(c == 1)
    def _():
        pltpu.sync_copy(acc.at[pl.ds(r0, rows_per_sub), pl.ds(0, 128)],
                        m1_hbm.at[pl.ds(r0, rows_per_sub), :])
        pltpu.sync_copy(acc.at[pl.ds(r0, rows_per_sub), pl.ds(128, 16)],
                        d1_hbm.at[pl.ds(r0, rows_per_sub), pl.ds(0, 16)])


def kernel(x, edge_index, edge_attr, W_e1, b_e1, W_e2, b_e2, Wq, bq, Wk, bk,
           Wv, bv, We, Wskip, bskip, Wbeta):
    n, d = x.shape
    e_cnt = edge_attr.shape[0]
    hc = Wq.shape[1]


    fold = pl.pallas_call(
        _fold_body,
        out_shape=(jax.ShapeDtypeStruct((d, d), jnp.float32),
                   jax.ShapeDtypeStruct((1, d), jnp.float32)),
    )
    w2e, b2e = fold(W_e2, We, b_e2.reshape(1, d))

    nblk = 2000
    node = pl.pallas_call(
        _node_body,
        grid=(n // nblk,),
        in_specs=[pl.BlockSpec((nblk, d), lambda i: (i, 0))] +
                 [pl.BlockSpec((d, hc), lambda i: (0, 0)),
                  pl.BlockSpec((1, hc), lambda i: (0, 0))] * 4,
        out_specs=[pl.BlockSpec((nblk, hc), lambda i: (i, 0))] * 4,
        out_shape=[jax.ShapeDtypeStruct((n, hc), jnp.float32)] * 4,
    )
    q, k, v, xr = node(x, Wq, bq.reshape(1, hc), Wk, bk.reshape(1, hc),
                       Wv, bv.reshape(1, hc), Wskip, bskip.reshape(1, hc))

    eblk = 4000
    edim = edge_attr.shape[1]
    edge = pl.pallas_call(
        _edge_body,
        grid=(e_cnt // eblk,),
        in_specs=[pl.BlockSpec((eblk, edim), lambda i: (i, 0)),
                  pl.BlockSpec((edim, d), lambda i: (0, 0)),
                  pl.BlockSpec((1, d), lambda i: (0, 0)),
                  pl.BlockSpec((d, d), lambda i: (0, 0)),
                  pl.BlockSpec((1, d), lambda i: (0, 0))],
        out_specs=pl.BlockSpec((eblk, d), lambda i: (i, 0)),
        out_shape=jax.ShapeDtypeStruct((e_cnt, d), jnp.float32),
    )
    e = edge(edge_attr, W_e1, b_e1.reshape(1, d), w2e, b2e)

    src = edge_index[0].astype(jnp.int32)
    dst = edge_index[1].astype(jnp.int32)

    mesh = plsc.VectorSubcoreMesh(core_axis_name="c", subcore_axis_name="s")
    sc = pl.kernel(
        functools.partial(_sc_body, n, e_cnt),
        out_type=(jax.ShapeDtypeStruct((n, d), jnp.float32),
                  jax.ShapeDtypeStruct((n, d), jnp.float32),
                  jax.ShapeDtypeStruct((n, d), jnp.float32),
                  jax.ShapeDtypeStruct((n, d), jnp.float32)),
        mesh=mesh,
        scratch_types=[
            pltpu.VMEM((_CHUNK,), jnp.int32),
            pltpu.VMEM((_CHUNK,), jnp.int32),
            pltpu.VMEM((_CHUNK, d), jnp.float32),
            pltpu.VMEM((_CHUNK, d), jnp.float32),
            pltpu.VMEM((_CHUNK, d), jnp.float32),
            pltpu.VMEM((_CHUNK, d), jnp.float32),
            pltpu.VMEM((_CHUNK, _ROW), jnp.float32),
            pltpu.VMEM_SHARED((n, _ROW), jnp.float32),
            pltpu.SemaphoreType.DMA,
            pltpu.SemaphoreType.DMA,
            pltpu.SemaphoreType.DMA,
            pltpu.SemaphoreType.DMA,
        ],
        compiler_params=pltpu.CompilerParams(use_tc_tiling_on_sc=False,
                                             needs_layout_passes=False),
    )
    m0, m1, d0, d1 = sc(q, k, v, e, src, dst)

    final = pl.pallas_call(
        _final_body,
        grid=(n // nblk,),
        in_specs=[pl.BlockSpec((nblk, hc), lambda i: (i, 0)),
                  pl.BlockSpec((nblk, hc), lambda i: (i, 0)),
                  pl.BlockSpec((nblk, hc), lambda i: (i, 0)),
                  pl.BlockSpec((nblk, hc), lambda i: (i, 0)),
                  pl.BlockSpec((nblk, hc), lambda i: (i, 0)),
                  pl.BlockSpec((3 * hc, 1), lambda i: (0, 0))],
        out_specs=pl.BlockSpec((nblk, hc), lambda i: (i, 0)),
        out_shape=jax.ShapeDtypeStruct((n, hc), jnp.float32),
    )
    return final(m0, m1, d0, d1, xr, Wbeta)
